# baseline (speedup 1.0000x reference)
import functools

import jax
import jax.numpy as jnp
from jax import lax
from jax.experimental import pallas as pl
from jax.experimental.pallas import tpu as pltpu
from jax.experimental.pallas import tpu_sc as plsc

N_NODES = 10000
N_EDGES = 320000
D = 128

NC = 2
NS = 16
NW = NC * NS
B = 128
NCHUNK = N_EDGES // B
CPT = 78
TB = 16
TAIL0 = NW * CPT * B
NP = 10240
RPT = NP // NS
L = 16


def _sc_aggregate(x_hbm, ei_hbm, src_hbm, dst_hbm,
                  acc_out, cnt_out,
                  eiA, eiB, rowsA, rowsB,
                  srcT, dstT, rowsT, tab_sh,
                  semGA, semGB, semI):
    cid = lax.axis_index("c")
    sid = lax.axis_index("s")
    wid = sid * NC + cid
    r0 = sid * RPT
    lo = wid * CPT

    def _set_rows(ref, val):
        def _row(i, carry):
            def _col(j, carry2):
                ref[i, pl.ds(j * L, L)] = jnp.full((L,), val, jnp.float32)
                return carry2
            lax.fori_loop(0, D // L, _col, 0)
            return carry
        lax.fori_loop(0, B, _row, 0)

    def _zero_table(zbuf):
        for k in range(RPT // B):
            pltpu.sync_copy(zbuf, tab_sh.at[pl.ds(r0 + k * B, B)])

    def _dump_table(out_ref, sbuf):
        for k in range(RPT // B):
            pltpu.sync_copy(tab_sh.at[pl.ds(r0 + k * B, B)], sbuf)
            pltpu.sync_copy(sbuf, out_ref.at[cid, pl.ds(r0 + k * B, B)])

    _set_rows(rowsA, 0.0)
    _zero_table(rowsA)
    plsc.subcore_barrier()

    bufs = [(eiA, rowsA, semGA), (eiB, rowsB, semGB)]

    pltpu.sync_copy(ei_hbm.at[:, pl.ds(lo * B, B)], eiA)
    pltpu.async_copy(x_hbm.at[eiA.at[0]], rowsA, semGA)
    pltpu.async_copy(ei_hbm.at[:, pl.ds((lo + 1) * B, B)], eiB, semI)

    def _chunk_step(p, k, issue_gather, prefetch_idx):
        ep, rp, gp = bufs[p]
        eq, rq, gq = bufs[1 - p]
        if issue_gather:
            pltpu.make_async_copy(
                ei_hbm.at[:, pl.ds((lo + k + 1) * B, B)], eq, semI).wait()
            pltpu.async_copy(x_hbm.at[eq.at[0]], rq, gq)
        pltpu.make_async_copy(x_hbm.at[ep.at[0]], rp, gp).wait()
        pltpu.sync_copy(rp, tab_sh.at[ep.at[1]], add=True)
        if prefetch_idx:
            pltpu.async_copy(ei_hbm.at[:, pl.ds((lo + k + 2) * B, B)], ep, semI)

    def _pair(t, carry):
        k = t * 2
        _chunk_step(0, k, True, True)
        _chunk_step(1, k + 1, True, True)
        return carry

    lax.fori_loop(0, CPT // 2 - 1, _pair, 0)
    _chunk_step(0, CPT - 2, True, False)
    _chunk_step(1, CPT - 1, False, False)

    tb = TAIL0 + wid * TB
    pltpu.sync_copy(src_hbm.at[pl.ds(tb, TB)], srcT)
    pltpu.sync_copy(dst_hbm.at[pl.ds(tb, TB)], dstT)
    pltpu.async_copy(x_hbm.at[srcT], rowsT, semGA).wait()
    pltpu.sync_copy(rowsT, tab_sh.at[dstT], add=True)

    plsc.subcore_barrier()
    _dump_table(acc_out, rowsA)

    _set_rows(rowsA, 0.0)
    _set_rows(rowsB, 1.0)
    _zero_table(rowsA)
    plsc.subcore_barrier()

    pltpu.sync_copy(ei_hbm.at[:, pl.ds(lo * B, B)], eiA)
    pltpu.async_copy(ei_hbm.at[:, pl.ds((lo + 1) * B, B)], eiB, semI)

    def _cnt_step(p, k, wait_idx, prefetch_idx):
        ep = bufs[p][0]
        eq = bufs[1 - p][0]
        if wait_idx:
            pltpu.make_async_copy(
                ei_hbm.at[:, pl.ds((lo + k + 1) * B, B)], eq, semI).wait()
        pltpu.sync_copy(rowsB, tab_sh.at[ep.at[1]], add=True)
        if prefetch_idx:
            pltpu.async_copy(ei_hbm.at[:, pl.ds((lo + k + 2) * B, B)], ep, semI)

    def _cnt_pair(t, carry):
        k = t * 2
        _cnt_step(0, k, True, True)
        _cnt_step(1, k + 1, True, True)
        return carry

    lax.fori_loop(0, CPT // 2 - 1, _cnt_pair, 0)
    _cnt_step(0, CPT - 2, True, False)
    _cnt_step(1, CPT - 1, False, False)

    pltpu.sync_copy(dst_hbm.at[pl.ds(tb, TB)], dstT)
    ones_t = rowsT
    def _fill_t(i, carry):
        def _col(j, carry2):
            ones_t[i, pl.ds(j * L, L)] = jnp.full((L,), 1.0, jnp.float32)
            return carry2
        lax.fori_loop(0, D // L, _col, 0)
        return carry
    lax.fori_loop(0, TB, _fill_t, 0)
    pltpu.sync_copy(ones_t, tab_sh.at[dstT], add=True)

    plsc.subcore_barrier()
    _dump_table(cnt_out, rowsA)


@functools.cache
def _sc_call():
    mesh = plsc.VectorSubcoreMesh(core_axis_name="c", subcore_axis_name="s",
                                  num_cores=NC, num_subcores=NS)
    return pl.kernel(
        _sc_aggregate,
        out_type=(
            jax.ShapeDtypeStruct((NC, NP, D), jnp.float32),
            jax.ShapeDtypeStruct((NC, NP, D), jnp.float32),
        ),
        mesh=mesh,
        scratch_types=[
            pltpu.VMEM((2, B), jnp.int32),
            pltpu.VMEM((2, B), jnp.int32),
            pltpu.VMEM((B, D), jnp.float32),
            pltpu.VMEM((B, D), jnp.float32),
            pltpu.VMEM((TB,), jnp.int32),
            pltpu.VMEM((TB,), jnp.int32),
            pltpu.VMEM((TB, D), jnp.float32),
            pltpu.VMEM_SHARED((NP, D), jnp.float32),
            pltpu.SemaphoreType.DMA,
            pltpu.SemaphoreType.DMA,
            pltpu.SemaphoreType.DMA,
        ],
    )


_TC_ROWS = 1000


def _tc_finish(acc_ref, cnt_ref, w_ref, b_ref, out_ref):
    s = acc_ref[0] + acc_ref[1]
    c = cnt_ref[0, :, 0:1] + cnt_ref[1, :, 0:1]
    m = s / jnp.maximum(c, 1.0)
    y = lax.dot_general(m, w_ref[...], (((1,), (1,)), ((), ())),
                        preferred_element_type=jnp.float32)
    out_ref[...] = jnp.where(c > 0.0, y + b_ref[...], 0.0)


def kernel(x, edge_index, W, b):
    src = edge_index[0]
    dst = edge_index[1]
    acc, cnt = _sc_call()(x, edge_index, src, dst)

    out = pl.pallas_call(
        _tc_finish,
        grid=(N_NODES // _TC_ROWS,),
        in_specs=[
            pl.BlockSpec((NC, _TC_ROWS, D), lambda i: (0, i, 0)),
            pl.BlockSpec((NC, _TC_ROWS, D), lambda i: (0, i, 0)),
            pl.BlockSpec((D, D), lambda i: (0, 0)),
            pl.BlockSpec((1, D), lambda i: (0, 0)),
        ],
        out_specs=pl.BlockSpec((_TC_ROWS, D), lambda i: (i, 0)),
        out_shape=jax.ShapeDtypeStruct((N_NODES, D), jnp.float32),
    )(acc, cnt, W, b.reshape(1, D))
    return out


# speedup vs baseline: 1.0125x; 1.0125x over previous
import functools

import jax
import jax.numpy as jnp
from jax import lax
from jax.experimental import pallas as pl
from jax.experimental.pallas import tpu as pltpu
from jax.experimental.pallas import tpu_sc as plsc

N_NODES = 10000
N_EDGES = 320000
D = 128

NC = 2
NS = 16
NW = NC * NS
B = 128
NCHUNK = N_EDGES // B
CPT = 78
TB = 16
TAIL0 = NW * CPT * B
NP = 10240
RPT = NP // NS
L = 16


def _sc_aggregate(x_hbm, src_hbm, dst_hbm,
                  acc_out, cnt_out,
                  srcA, srcB, dstA, dstB, rowsA, rowsB,
                  srcT, dstT, rowsT, tab_sh,
                  semGA, semGA2, semGB, semGB2, semI):
    cid = lax.axis_index("c")
    sid = lax.axis_index("s")
    wid = sid * NC + cid
    r0 = sid * RPT
    lo = wid * CPT

    def _set_rows(ref, val):
        def _row(i, carry):
            def _col(j, carry2):
                ref[i, pl.ds(j * L, L)] = jnp.full((L,), val, jnp.float32)
                return carry2
            lax.fori_loop(0, D // L, _col, 0)
            return carry
        lax.fori_loop(0, B, _row, 0)

    def _zero_table(zbuf):
        for k in range(RPT // B):
            pltpu.sync_copy(zbuf, tab_sh.at[pl.ds(r0 + k * B, B)])

    def _dump_table(out_ref, sbuf):
        for k in range(RPT // B):
            pltpu.sync_copy(tab_sh.at[pl.ds(r0 + k * B, B)], sbuf)
            pltpu.sync_copy(sbuf, out_ref.at[cid, pl.ds(r0 + k * B, B)])

    _set_rows(rowsA, 0.0)
    _zero_table(rowsA)
    plsc.subcore_barrier()

    bufs = [(srcA, dstA, rowsA, semGA, semGA2), (srcB, dstB, rowsB, semGB, semGB2)]
    H = B // 2

    def _issue_gather2(s_ref, r_ref, g1, g2):
        pltpu.async_copy(x_hbm.at[s_ref.at[pl.ds(0, H)]], r_ref.at[pl.ds(0, H)], g1)
        pltpu.async_copy(x_hbm.at[s_ref.at[pl.ds(H, H)]], r_ref.at[pl.ds(H, H)], g2)

    def _wait_gather2(s_ref, r_ref, g1, g2):
        pltpu.make_async_copy(x_hbm.at[s_ref.at[pl.ds(0, H)]], r_ref.at[pl.ds(0, H)], g1).wait()
        pltpu.make_async_copy(x_hbm.at[s_ref.at[pl.ds(H, H)]], r_ref.at[pl.ds(H, H)], g2).wait()

    pltpu.sync_copy(src_hbm.at[pl.ds(lo * B, B)], srcA)
    pltpu.sync_copy(dst_hbm.at[pl.ds(lo * B, B)], dstA)
    _issue_gather2(srcA, rowsA, semGA, semGA2)
    pltpu.async_copy(src_hbm.at[pl.ds((lo + 1) * B, B)], srcB, semI)
    pltpu.async_copy(dst_hbm.at[pl.ds((lo + 1) * B, B)], dstB, semI)

    def _chunk_step(p, k, issue_gather, prefetch_idx):
        sp, dp, rp, gp, gp2 = bufs[p]
        sq, dq, rq, gq, gq2 = bufs[1 - p]
        if issue_gather:
            pltpu.make_async_copy(
                src_hbm.at[pl.ds((lo + k + 1) * B, B)], sq, semI).wait()
            pltpu.make_async_copy(
                dst_hbm.at[pl.ds((lo + k + 1) * B, B)], dq, semI).wait()
            _issue_gather2(sq, rq, gq, gq2)
        _wait_gather2(sp, rp, gp, gp2)
        pltpu.sync_copy(rp, tab_sh.at[dp], add=True)
        if prefetch_idx:
            pltpu.async_copy(src_hbm.at[pl.ds((lo + k + 2) * B, B)], sp, semI)
            pltpu.async_copy(dst_hbm.at[pl.ds((lo + k + 2) * B, B)], dp, semI)

    def _pair(t, carry):
        k = t * 2
        _chunk_step(0, k, True, True)
        _chunk_step(1, k + 1, True, True)
        return carry

    lax.fori_loop(0, CPT // 2 - 1, _pair, 0)
    _chunk_step(0, CPT - 2, True, False)
    _chunk_step(1, CPT - 1, False, False)

    tb = TAIL0 + wid * TB
    pltpu.sync_copy(src_hbm.at[pl.ds(tb, TB)], srcT)
    pltpu.sync_copy(dst_hbm.at[pl.ds(tb, TB)], dstT)
    pltpu.async_copy(x_hbm.at[srcT], rowsT, semGA).wait()
    pltpu.sync_copy(rowsT, tab_sh.at[dstT], add=True)

    plsc.subcore_barrier()
    _dump_table(acc_out, rowsA)

    _set_rows(rowsA, 0.0)
    _set_rows(rowsB, 1.0)
    _zero_table(rowsA)
    plsc.subcore_barrier()

    pltpu.sync_copy(dst_hbm.at[pl.ds(lo * B, B)], dstA)
    pltpu.async_copy(dst_hbm.at[pl.ds((lo + 1) * B, B)], dstB, semI)

    def _cnt_step(p, k, wait_idx, prefetch_idx):
        dp = bufs[p][1]
        dq = bufs[1 - p][1]
        if wait_idx:
            pltpu.make_async_copy(
                dst_hbm.at[pl.ds((lo + k + 1) * B, B)], dq, semI).wait()
        pltpu.sync_copy(rowsB, tab_sh.at[dp], add=True)
        if prefetch_idx:
            pltpu.async_copy(dst_hbm.at[pl.ds((lo + k + 2) * B, B)], dp, semI)

    def _cnt_pair(t, carry):
        k = t * 2
        _cnt_step(0, k, True, True)
        _cnt_step(1, k + 1, True, True)
        return carry

    lax.fori_loop(0, CPT // 2 - 1, _cnt_pair, 0)
    _cnt_step(0, CPT - 2, True, False)
    _cnt_step(1, CPT - 1, False, False)

    pltpu.sync_copy(dst_hbm.at[pl.ds(tb, TB)], dstT)
    ones_t = rowsT
    def _fill_t(i, carry):
        def _col(j, carry2):
            ones_t[i, pl.ds(j * L, L)] = jnp.full((L,), 1.0, jnp.float32)
            return carry2
        lax.fori_loop(0, D // L, _col, 0)
        return carry
    lax.fori_loop(0, TB, _fill_t, 0)
    pltpu.sync_copy(ones_t, tab_sh.at[dstT], add=True)

    plsc.subcore_barrier()
    _dump_table(cnt_out, rowsA)


@functools.cache
def _sc_call():
    mesh = plsc.VectorSubcoreMesh(core_axis_name="c", subcore_axis_name="s",
                                  num_cores=NC, num_subcores=NS)
    return pl.kernel(
        _sc_aggregate,
        out_type=(
            jax.ShapeDtypeStruct((NC, NP, D), jnp.float32),
            jax.ShapeDtypeStruct((NC, NP, D), jnp.float32),
        ),
        mesh=mesh,
        scratch_types=[
            pltpu.VMEM((B,), jnp.int32),
            pltpu.VMEM((B,), jnp.int32),
            pltpu.VMEM((B,), jnp.int32),
            pltpu.VMEM((B,), jnp.int32),
            pltpu.VMEM((B, D), jnp.float32),
            pltpu.VMEM((B, D), jnp.float32),
            pltpu.VMEM((TB,), jnp.int32),
            pltpu.VMEM((TB,), jnp.int32),
            pltpu.VMEM((TB, D), jnp.float32),
            pltpu.VMEM_SHARED((NP, D), jnp.float32),
            pltpu.SemaphoreType.DMA,
            pltpu.SemaphoreType.DMA,
            pltpu.SemaphoreType.DMA,
            pltpu.SemaphoreType.DMA,
            pltpu.SemaphoreType.DMA,
        ],
    )


_TC_ROWS = 1000


def _tc_finish(acc_ref, cnt_ref, w_ref, b_ref, out_ref):
    s = acc_ref[0] + acc_ref[1]
    c = cnt_ref[0, :, 0:1] + cnt_ref[1, :, 0:1]
    m = s / jnp.maximum(c, 1.0)
    y = lax.dot_general(m, w_ref[...], (((1,), (1,)), ((), ())),
                        preferred_element_type=jnp.float32)
    out_ref[...] = jnp.where(c > 0.0, y + b_ref[...], 0.0)


def kernel(x, edge_index, W, b):
    src = edge_index[0]
    dst = edge_index[1]
    acc, cnt = _sc_call()(x, src, dst)

    out = pl.pallas_call(
        _tc_finish,
        grid=(N_NODES // _TC_ROWS,),
        in_specs=[
            pl.BlockSpec((NC, _TC_ROWS, D), lambda i: (0, i, 0)),
            pl.BlockSpec((NC, _TC_ROWS, D), lambda i: (0, i, 0)),
            pl.BlockSpec((D, D), lambda i: (0, 0)),
            pl.BlockSpec((1, D), lambda i: (0, 0)),
        ],
        out_specs=pl.BlockSpec((_TC_ROWS, D), lambda i: (i, 0)),
        out_shape=jax.ShapeDtypeStruct((N_NODES, D), jnp.float32),
    )(acc, cnt, W, b.reshape(1, D))
    return out


# speedup vs baseline: 1.0402x; 1.0274x over previous
import functools

import jax
import jax.numpy as jnp
from jax import lax
from jax.experimental import pallas as pl
from jax.experimental.pallas import tpu as pltpu
from jax.experimental.pallas import tpu_sc as plsc

N_NODES = 10000
N_EDGES = 320000
D = 128

NC = 2
NS = 16
NW = NC * NS
B = 128
NCHUNK = N_EDGES // B
CPT = 78
TB = 16
TAIL0 = NW * CPT * B
NP = 10240
RPT = NP // NS
L = 16


def _sc_aggregate(x_hbm, src_hbm, dst_hbm,
                  acc_out, cnt_out,
                  srcA, srcB, dstA, dstB, rowsA, rowsB,
                  srcT, dstT, rowsT, tab_sh,
                  semGA, semGA2, semGB, semGB2, semI):
    cid = lax.axis_index("c")
    sid = lax.axis_index("s")
    wid = sid * NC + cid
    r0 = sid * RPT
    lo = wid * CPT

    def _set_rows(ref, val):
        def _row(i, carry):
            def _col(j, carry2):
                ref[i, pl.ds(j * L, L)] = jnp.full((L,), val, jnp.float32)
                return carry2
            lax.fori_loop(0, D // L, _col, 0)
            return carry
        lax.fori_loop(0, B, _row, 0)

    def _zero_table(zbuf):
        for k in range(RPT // B):
            pltpu.sync_copy(zbuf, tab_sh.at[pl.ds(r0 + k * B, B)])

    def _dump_table(out_ref, sbuf0, sbuf1, sem0, sem1):
        stg = [(sbuf0, sem0), (sbuf1, sem1)]
        for k in range(RPT // B):
            sb, sm = stg[k % 2]
            if k >= 2:
                pltpu.make_async_copy(
                    sb, out_ref.at[cid, pl.ds(r0 + (k - 2) * B, B)], sm).wait()
            pltpu.sync_copy(tab_sh.at[pl.ds(r0 + k * B, B)], sb)
            pltpu.async_copy(sb, out_ref.at[cid, pl.ds(r0 + k * B, B)], sm)
        for k in range(RPT // B - 2, RPT // B):
            sb, sm = stg[k % 2]
            pltpu.make_async_copy(
                sb, out_ref.at[cid, pl.ds(r0 + k * B, B)], sm).wait()

    _set_rows(rowsA, 0.0)
    _zero_table(rowsA)
    plsc.subcore_barrier()

    bufs = [(srcA, dstA, rowsA, semGA, semGA2), (srcB, dstB, rowsB, semGB, semGB2)]
    H = B // 2

    def _issue_gather2(s_ref, r_ref, g1, g2):
        pltpu.async_copy(x_hbm.at[s_ref.at[pl.ds(0, H)]], r_ref.at[pl.ds(0, H)], g1)
        pltpu.async_copy(x_hbm.at[s_ref.at[pl.ds(H, H)]], r_ref.at[pl.ds(H, H)], g2)

    def _wait_gather2(s_ref, r_ref, g1, g2):
        pltpu.make_async_copy(x_hbm.at[s_ref.at[pl.ds(0, H)]], r_ref.at[pl.ds(0, H)], g1).wait()
        pltpu.make_async_copy(x_hbm.at[s_ref.at[pl.ds(H, H)]], r_ref.at[pl.ds(H, H)], g2).wait()

    pltpu.sync_copy(src_hbm.at[pl.ds(lo * B, B)], srcA)
    pltpu.sync_copy(dst_hbm.at[pl.ds(lo * B, B)], dstA)
    _issue_gather2(srcA, rowsA, semGA, semGA2)
    pltpu.async_copy(src_hbm.at[pl.ds((lo + 1) * B, B)], srcB, semI)
    pltpu.async_copy(dst_hbm.at[pl.ds((lo + 1) * B, B)], dstB, semI)

    def _chunk_step(p, k, issue_gather, prefetch_idx):
        sp, dp, rp, gp, gp2 = bufs[p]
        sq, dq, rq, gq, gq2 = bufs[1 - p]
        if issue_gather:
            pltpu.make_async_copy(
                src_hbm.at[pl.ds((lo + k + 1) * B, B)], sq, semI).wait()
            pltpu.make_async_copy(
                dst_hbm.at[pl.ds((lo + k + 1) * B, B)], dq, semI).wait()
            _issue_gather2(sq, rq, gq, gq2)
        _wait_gather2(sp, rp, gp, gp2)
        pltpu.sync_copy(rp, tab_sh.at[dp], add=True)
        if prefetch_idx:
            pltpu.async_copy(src_hbm.at[pl.ds((lo + k + 2) * B, B)], sp, semI)
            pltpu.async_copy(dst_hbm.at[pl.ds((lo + k + 2) * B, B)], dp, semI)

    def _pair(t, carry):
        k = t * 2
        _chunk_step(0, k, True, True)
        _chunk_step(1, k + 1, True, True)
        return carry

    lax.fori_loop(0, CPT // 2 - 1, _pair, 0)
    _chunk_step(0, CPT - 2, True, False)
    _chunk_step(1, CPT - 1, False, False)

    tb = TAIL0 + wid * TB
    pltpu.sync_copy(src_hbm.at[pl.ds(tb, TB)], srcT)
    pltpu.sync_copy(dst_hbm.at[pl.ds(tb, TB)], dstT)
    pltpu.async_copy(x_hbm.at[srcT], rowsT, semGA).wait()
    pltpu.sync_copy(rowsT, tab_sh.at[dstT], add=True)

    plsc.subcore_barrier()
    _dump_table(acc_out, rowsA, rowsB, semGA, semGB)

    _set_rows(rowsB, 1.0)
    plsc.subcore_barrier()

    pltpu.sync_copy(dst_hbm.at[pl.ds(lo * B, B)], dstA)
    pltpu.async_copy(dst_hbm.at[pl.ds((lo + 1) * B, B)], dstB, semI)

    def _cnt_step(p, k, wait_idx, prefetch_idx):
        dp = bufs[p][1]
        dq = bufs[1 - p][1]
        if wait_idx:
            pltpu.make_async_copy(
                dst_hbm.at[pl.ds((lo + k + 1) * B, B)], dq, semI).wait()
        pltpu.sync_copy(rowsB, tab_sh.at[dp], add=True)
        if prefetch_idx:
            pltpu.async_copy(dst_hbm.at[pl.ds((lo + k + 2) * B, B)], dp, semI)

    def _cnt_pair(t, carry):
        k = t * 2
        _cnt_step(0, k, True, True)
        _cnt_step(1, k + 1, True, True)
        return carry

    lax.fori_loop(0, CPT // 2 - 1, _cnt_pair, 0)
    _cnt_step(0, CPT - 2, True, False)
    _cnt_step(1, CPT - 1, False, False)

    pltpu.sync_copy(dst_hbm.at[pl.ds(tb, TB)], dstT)
    ones_t = rowsT
    def _fill_t(i, carry):
        def _col(j, carry2):
            ones_t[i, pl.ds(j * L, L)] = jnp.full((L,), 1.0, jnp.float32)
            return carry2
        lax.fori_loop(0, D // L, _col, 0)
        return carry
    lax.fori_loop(0, TB, _fill_t, 0)
    pltpu.sync_copy(ones_t, tab_sh.at[dstT], add=True)

    plsc.subcore_barrier()
    _dump_table(cnt_out, rowsA, rowsB, semGA, semGB)


@functools.cache
def _sc_call():
    mesh = plsc.VectorSubcoreMesh(core_axis_name="c", subcore_axis_name="s",
                                  num_cores=NC, num_subcores=NS)
    return pl.kernel(
        _sc_aggregate,
        out_type=(
            jax.ShapeDtypeStruct((NC, NP, D), jnp.float32),
            jax.ShapeDtypeStruct((NC, NP, D), jnp.float32),
        ),
        mesh=mesh,
        scratch_types=[
            pltpu.VMEM((B,), jnp.int32),
            pltpu.VMEM((B,), jnp.int32),
            pltpu.VMEM((B,), jnp.int32),
            pltpu.VMEM((B,), jnp.int32),
            pltpu.VMEM((B, D), jnp.float32),
            pltpu.VMEM((B, D), jnp.float32),
            pltpu.VMEM((TB,), jnp.int32),
            pltpu.VMEM((TB,), jnp.int32),
            pltpu.VMEM((TB, D), jnp.float32),
            pltpu.VMEM_SHARED((NP, D), jnp.float32),
            pltpu.SemaphoreType.DMA,
            pltpu.SemaphoreType.DMA,
            pltpu.SemaphoreType.DMA,
            pltpu.SemaphoreType.DMA,
            pltpu.SemaphoreType.DMA,
        ],
    )


_TC_ROWS = 1000


def _tc_finish(acc_ref, cnt_ref, w_ref, b_ref, out_ref):
    s = acc_ref[0] + acc_ref[1]
    c = (cnt_ref[0, :, 0:1] - acc_ref[0, :, 0:1]) + \
        (cnt_ref[1, :, 0:1] - acc_ref[1, :, 0:1])
    m = s / jnp.maximum(c, 1.0)
    y = lax.dot_general(m, w_ref[...], (((1,), (1,)), ((), ())),
                        preferred_element_type=jnp.float32)
    out_ref[...] = jnp.where(c > 0.0, y + b_ref[...], 0.0)


def kernel(x, edge_index, W, b):
    src = edge_index[0]
    dst = edge_index[1]
    acc, cnt = _sc_call()(x, src, dst)

    out = pl.pallas_call(
        _tc_finish,
        grid=(N_NODES // _TC_ROWS,),
        in_specs=[
            pl.BlockSpec((NC, _TC_ROWS, D), lambda i: (0, i, 0)),
            pl.BlockSpec((NC, _TC_ROWS, D), lambda i: (0, i, 0)),
            pl.BlockSpec((D, D), lambda i: (0, 0)),
            pl.BlockSpec((1, D), lambda i: (0, 0)),
        ],
        out_specs=pl.BlockSpec((_TC_ROWS, D), lambda i: (i, 0)),
        out_shape=jax.ShapeDtypeStruct((N_NODES, D), jnp.float32),
    )(acc, cnt, W, b.reshape(1, D))
    return out


# speedup vs baseline: 1.0481x; 1.0075x over previous
import functools

import jax
import jax.numpy as jnp
from jax import lax
from jax.experimental import pallas as pl
from jax.experimental.pallas import tpu as pltpu
from jax.experimental.pallas import tpu_sc as plsc

N_NODES = 10000
N_EDGES = 320000
D = 128

NC = 2
NS = 16
NW = NC * NS
B = 128
NCHUNK = N_EDGES // B
CPT = 78
TB = 16
TAIL0 = NW * CPT * B
NP = 10240
RPT = NP // NS
L = 16


def _sc_aggregate(x_hbm, src_hbm, dst_hbm,
                  acc_out, cnt_out,
                  srcA, srcB, dstA, dstB, rowsA, rowsB,
                  srcT, dstT, rowsT, tab_sh,
                  semGA, semGA2, semGB, semGB2, semI, semT):
    cid = lax.axis_index("c")
    sid = lax.axis_index("s")
    wid = sid * NC + cid
    r0 = sid * RPT
    lo = wid * CPT

    def _set_rows(ref, val):
        def _row(i, carry):
            def _col(j, carry2):
                ref[i, pl.ds(j * L, L)] = jnp.full((L,), val, jnp.float32)
                return carry2
            lax.fori_loop(0, D // L, _col, 0)
            return carry
        lax.fori_loop(0, B, _row, 0)

    def _zero_table(zbuf):
        for k in range(RPT // B):
            pltpu.sync_copy(zbuf, tab_sh.at[pl.ds(r0 + k * B, B)])

    def _dump_table(out_ref, sbuf0, sbuf1, sem0, sem1):
        stg = [(sbuf0, sem0), (sbuf1, sem1)]
        for k in range(RPT // B):
            sb, sm = stg[k % 2]
            if k >= 2:
                pltpu.make_async_copy(
                    sb, out_ref.at[cid, pl.ds(r0 + (k - 2) * B, B)], sm).wait()
            pltpu.sync_copy(tab_sh.at[pl.ds(r0 + k * B, B)], sb)
            pltpu.async_copy(sb, out_ref.at[cid, pl.ds(r0 + k * B, B)], sm)
        for k in range(RPT // B - 2, RPT // B):
            sb, sm = stg[k % 2]
            pltpu.make_async_copy(
                sb, out_ref.at[cid, pl.ds(r0 + k * B, B)], sm).wait()

    _set_rows(rowsA, 0.0)
    _zero_table(rowsA)
    plsc.subcore_barrier()

    bufs = [(srcA, dstA, rowsA, semGA, semGA2), (srcB, dstB, rowsB, semGB, semGB2)]
    H = B // 2

    def _issue_gather2(s_ref, r_ref, g1, g2):
        pltpu.async_copy(x_hbm.at[s_ref.at[pl.ds(0, H)]], r_ref.at[pl.ds(0, H)], g1)
        pltpu.async_copy(x_hbm.at[s_ref.at[pl.ds(H, H)]], r_ref.at[pl.ds(H, H)], g2)

    def _wait_gather2(s_ref, r_ref, g1, g2):
        pltpu.make_async_copy(x_hbm.at[s_ref.at[pl.ds(0, H)]], r_ref.at[pl.ds(0, H)], g1).wait()
        pltpu.make_async_copy(x_hbm.at[s_ref.at[pl.ds(H, H)]], r_ref.at[pl.ds(H, H)], g2).wait()

    pltpu.sync_copy(src_hbm.at[pl.ds(lo * B, B)], srcA)
    pltpu.sync_copy(dst_hbm.at[pl.ds(lo * B, B)], dstA)
    _issue_gather2(srcA, rowsA, semGA, semGA2)
    pltpu.async_copy(src_hbm.at[pl.ds((lo + 1) * B, B)], srcB, semI)
    pltpu.async_copy(dst_hbm.at[pl.ds((lo + 1) * B, B)], dstB, semI)

    def _chunk_step(p, k, issue_gather, prefetch_idx):
        sp, dp, rp, gp, gp2 = bufs[p]
        sq, dq, rq, gq, gq2 = bufs[1 - p]
        if issue_gather:
            pltpu.make_async_copy(
                src_hbm.at[pl.ds((lo + k + 1) * B, B)], sq, semI).wait()
            pltpu.make_async_copy(
                dst_hbm.at[pl.ds((lo + k + 1) * B, B)], dq, semI).wait()
            _issue_gather2(sq, rq, gq, gq2)
        _wait_gather2(sp, rp, gp, gp2)
        pltpu.sync_copy(rp, tab_sh.at[dp], add=True)
        if prefetch_idx:
            pltpu.async_copy(src_hbm.at[pl.ds((lo + k + 2) * B, B)], sp, semI)
            pltpu.async_copy(dst_hbm.at[pl.ds((lo + k + 2) * B, B)], dp, semI)

    def _pair(t, carry):
        k = t * 2
        _chunk_step(0, k, True, True)
        _chunk_step(1, k + 1, True, True)
        return carry

    lax.fori_loop(0, CPT // 2 - 1, _pair, 0)
    tb = TAIL0 + wid * TB
    pltpu.async_copy(src_hbm.at[pl.ds(tb, TB)], srcT, semT)
    pltpu.async_copy(dst_hbm.at[pl.ds(tb, TB)], dstT, semT)
    _chunk_step(0, CPT - 2, True, False)
    _chunk_step(1, CPT - 1, False, False)

    pltpu.make_async_copy(src_hbm.at[pl.ds(tb, TB)], srcT, semT).wait()
    pltpu.make_async_copy(dst_hbm.at[pl.ds(tb, TB)], dstT, semT).wait()
    pltpu.async_copy(x_hbm.at[srcT], rowsT, semGA).wait()
    pltpu.sync_copy(rowsT, tab_sh.at[dstT], add=True)

    pltpu.async_copy(dst_hbm.at[pl.ds(lo * B, B)], dstA, semI)
    pltpu.async_copy(dst_hbm.at[pl.ds((lo + 1) * B, B)], dstB, semI)

    plsc.subcore_barrier()
    _dump_table(acc_out, rowsA, rowsB, semGA, semGB)

    _set_rows(rowsB, 1.0)
    plsc.subcore_barrier()

    pltpu.make_async_copy(dst_hbm.at[pl.ds(lo * B, B)], dstA, semI).wait()

    def _cnt_step(p, k, wait_idx, prefetch_idx):
        dp = bufs[p][1]
        dq = bufs[1 - p][1]
        if wait_idx:
            pltpu.make_async_copy(
                dst_hbm.at[pl.ds((lo + k + 1) * B, B)], dq, semI).wait()
        pltpu.sync_copy(rowsB, tab_sh.at[dp], add=True)
        if prefetch_idx:
            pltpu.async_copy(dst_hbm.at[pl.ds((lo + k + 2) * B, B)], dp, semI)

    def _cnt_pair(t, carry):
        k = t * 2
        _cnt_step(0, k, True, True)
        _cnt_step(1, k + 1, True, True)
        return carry

    lax.fori_loop(0, CPT // 2 - 1, _cnt_pair, 0)
    _cnt_step(0, CPT - 2, True, False)
    _cnt_step(1, CPT - 1, False, False)

    pltpu.sync_copy(dst_hbm.at[pl.ds(tb, TB)], dstT)
    ones_t = rowsT
    def _fill_t(i, carry):
        def _col(j, carry2):
            ones_t[i, pl.ds(j * L, L)] = jnp.full((L,), 1.0, jnp.float32)
            return carry2
        lax.fori_loop(0, D // L, _col, 0)
        return carry
    lax.fori_loop(0, TB, _fill_t, 0)
    pltpu.sync_copy(ones_t, tab_sh.at[dstT], add=True)

    plsc.subcore_barrier()
    _dump_table(cnt_out, rowsA, rowsB, semGA, semGB)


@functools.cache
def _sc_call():
    mesh = plsc.VectorSubcoreMesh(core_axis_name="c", subcore_axis_name="s",
                                  num_cores=NC, num_subcores=NS)
    return pl.kernel(
        _sc_aggregate,
        out_type=(
            jax.ShapeDtypeStruct((NC, NP, D), jnp.float32),
            jax.ShapeDtypeStruct((NC, NP, D), jnp.float32),
        ),
        mesh=mesh,
        scratch_types=[
            pltpu.VMEM((B,), jnp.int32),
            pltpu.VMEM((B,), jnp.int32),
            pltpu.VMEM((B,), jnp.int32),
            pltpu.VMEM((B,), jnp.int32),
            pltpu.VMEM((B, D), jnp.float32),
            pltpu.VMEM((B, D), jnp.float32),
            pltpu.VMEM((TB,), jnp.int32),
            pltpu.VMEM((TB,), jnp.int32),
            pltpu.VMEM((TB, D), jnp.float32),
            pltpu.VMEM_SHARED((NP, D), jnp.float32),
            pltpu.SemaphoreType.DMA,
            pltpu.SemaphoreType.DMA,
            pltpu.SemaphoreType.DMA,
            pltpu.SemaphoreType.DMA,
            pltpu.SemaphoreType.DMA,
            pltpu.SemaphoreType.DMA,
        ],
    )


_TC_ROWS = 1000


def _tc_finish(acc_ref, cnt_ref, w_ref, b_ref, out_ref):
    s = acc_ref[0] + acc_ref[1]
    c = (cnt_ref[0, :, 0:1] - acc_ref[0, :, 0:1]) + \
        (cnt_ref[1, :, 0:1] - acc_ref[1, :, 0:1])
    m = s / jnp.maximum(c, 1.0)
    y = lax.dot_general(m, w_ref[...], (((1,), (1,)), ((), ())),
                        preferred_element_type=jnp.float32)
    out_ref[...] = jnp.where(c > 0.0, y + b_ref[...], 0.0)


def kernel(x, edge_index, W, b):
    src = edge_index[0]
    dst = edge_index[1]
    acc, cnt = _sc_call()(x, src, dst)

    out = pl.pallas_call(
        _tc_finish,
        grid=(N_NODES // _TC_ROWS,),
        in_specs=[
            pl.BlockSpec((NC, _TC_ROWS, D), lambda i: (0, i, 0)),
            pl.BlockSpec((NC, _TC_ROWS, D), lambda i: (0, i, 0)),
            pl.BlockSpec((D, D), lambda i: (0, 0)),
            pl.BlockSpec((1, D), lambda i: (0, 0)),
        ],
        out_specs=pl.BlockSpec((_TC_ROWS, D), lambda i: (i, 0)),
        out_shape=jax.ShapeDtypeStruct((N_NODES, D), jnp.float32),
    )(acc, cnt, W, b.reshape(1, D))
    return out
